# speedup vs baseline: 7.4979x; 7.4979x over previous
import jax
import jax.numpy as jnp
from jax.experimental import pallas as pl
from jax.experimental.pallas import tpu as pltpu

N = 32768
C = 128
B = 16
OUT = 128
TILE = 1024
NT = N // TILE
SEG = B + 1
SEGP = 32


def _body(off_smem, feat_ref, w1t_ref, b1_ref, gamma_ref, beta_ref,
          wmx_ref, wmn_ref, bo_ref, out_ref,
          h_buf, colsum, colsumsq, scale_r, shift_r,
          segmax, segmin, segcnt, segsum):
    p = pl.program_id(0)
    t = pl.program_id(1)
    r0 = t * TILE

    @pl.when(p == 0)
    def _phase0():
        @pl.when(t == 0)
        def _init0():
            colsum[...] = jnp.zeros_like(colsum)
            colsumsq[...] = jnp.zeros_like(colsumsq)
            segmax[...] = jnp.full_like(segmax, -jnp.inf)
            segmin[...] = jnp.full_like(segmin, jnp.inf)
            segcnt[...] = jnp.zeros_like(segcnt)

        h = jnp.dot(feat_ref[...], w1t_ref[...],
                    preferred_element_type=jnp.float32) + b1_ref[...]
        h_buf[t] = h
        colsum[...] += jnp.sum(h, axis=0, keepdims=True)
        colsumsq[...] += jnp.sum(h * h, axis=0, keepdims=True)

        b_lo = jnp.int32(0)
        b_hi = jnp.int32(0)
        for j in range(B):
            oj = off_smem[j]
            b_lo += (oj <= r0).astype(jnp.int32)
            b_hi += (oj <= r0 + TILE - 1).astype(jnp.int32)

        rows = r0 + jax.lax.broadcasted_iota(jnp.int32, (TILE, 1), 0)
        seg_iota = jax.lax.broadcasted_iota(jnp.int32, (SEGP, 1), 0)

        def seg_body(s, carry):
            start = jnp.where(s == 0, 0, off_smem[jnp.maximum(s - 1, 0)])
            end = jnp.where(s == B, N, off_smem[jnp.minimum(s, B - 1)])
            m = (rows >= start) & (rows < end)
            hmax = jnp.max(jnp.where(m, h, -jnp.inf), axis=0, keepdims=True)
            hmin = jnp.min(jnp.where(m, h, jnp.inf), axis=0, keepdims=True)
            c = jnp.sum(m.astype(jnp.float32), axis=0, keepdims=True)
            sel = seg_iota == s
            segmax[...] = jnp.where(sel, jnp.maximum(segmax[...], hmax), segmax[...])
            segmin[...] = jnp.where(sel, jnp.minimum(segmin[...], hmin), segmin[...])
            segcnt[...] = jnp.where(sel, segcnt[...] + c, segcnt[...])
            return carry

        jax.lax.fori_loop(b_lo, b_hi + 1, seg_body, 0)

    @pl.when(p == 1)
    def _phase1():
        @pl.when(t == 0)
        def _init1():
            mu = colsum[...] * (1.0 / N)
            var = colsumsq[...] * (1.0 / N) - mu * mu
            sc = gamma_ref[...] * jax.lax.rsqrt(var + 1e-5)
            scale_r[...] = sc
            shift_r[...] = beta_ref[...] - mu * sc
            segsum[...] = jnp.zeros_like(segsum)

        h = h_buf[t]
        hn = jnp.maximum(h * scale_r[...] + shift_r[...], 0.0)

        rows_row = r0 + jax.lax.broadcasted_iota(jnp.int32, (1, TILE), 1)
        bvec = jnp.zeros((1, TILE), jnp.int32)
        for j in range(B):
            bvec += (off_smem[j] <= rows_row).astype(jnp.int32)
        onehot_t = (jax.lax.broadcasted_iota(jnp.int32, (SEGP, TILE), 0)
                    == bvec).astype(jnp.float32)
        segsum[...] += jnp.dot(onehot_t, hn, preferred_element_type=jnp.float32)

        @pl.when(t == NT - 1)
        def _finalize():
            cnt = segcnt[...]
            mean = segsum[...] / jnp.maximum(cnt, 1.0)
            raw = jnp.where(scale_r[...] >= 0.0, segmax[...], segmin[...])
            mx = jnp.maximum(raw * scale_r[...] + shift_r[...], 0.0)
            mx = jnp.where(cnt > 0.0, mx, 0.0)
            out_ref[...] = (jnp.dot(mx, wmx_ref[...],
                                    preferred_element_type=jnp.float32)
                            + jnp.dot(mean, wmn_ref[...],
                                      preferred_element_type=jnp.float32)
                            + bo_ref[...])


def kernel(feat, offset, W1, b1, gamma, beta, Wo, bo):
    w1t = W1.T
    wmx = Wo[:, :C].T
    wmn = Wo[:, C:].T
    b1r = b1.reshape(1, C)
    gr = gamma.reshape(1, C)
    br = beta.reshape(1, C)
    bor = bo.reshape(1, OUT)

    grid_spec = pltpu.PrefetchScalarGridSpec(
        num_scalar_prefetch=1,
        grid=(2, NT),
        in_specs=[
            pl.BlockSpec((TILE, C),
                         lambda p, t, off: (jnp.where(p == 0, t, NT - 1), 0)),
            pl.BlockSpec((C, C), lambda p, t, off: (0, 0)),
            pl.BlockSpec((1, C), lambda p, t, off: (0, 0)),
            pl.BlockSpec((1, C), lambda p, t, off: (0, 0)),
            pl.BlockSpec((1, C), lambda p, t, off: (0, 0)),
            pl.BlockSpec((C, OUT), lambda p, t, off: (0, 0)),
            pl.BlockSpec((C, OUT), lambda p, t, off: (0, 0)),
            pl.BlockSpec((1, OUT), lambda p, t, off: (0, 0)),
        ],
        out_specs=pl.BlockSpec((SEGP, OUT), lambda p, t, off: (0, 0)),
        scratch_shapes=[
            pltpu.VMEM((NT, TILE, C), jnp.float32),
            pltpu.VMEM((1, C), jnp.float32),
            pltpu.VMEM((1, C), jnp.float32),
            pltpu.VMEM((1, C), jnp.float32),
            pltpu.VMEM((1, C), jnp.float32),
            pltpu.VMEM((SEGP, C), jnp.float32),
            pltpu.VMEM((SEGP, C), jnp.float32),
            pltpu.VMEM((SEGP, C), jnp.float32),
            pltpu.VMEM((SEGP, C), jnp.float32),
        ],
    )

    out = pl.pallas_call(
        _body,
        grid_spec=grid_spec,
        out_shape=jax.ShapeDtypeStruct((SEGP, OUT), jnp.float32),
        compiler_params=pltpu.CompilerParams(
            dimension_semantics=("arbitrary", "arbitrary")),
    )(offset, feat, w1t, b1r, gr, br, wmx, wmn, bor)
    return out[:SEG]
